# baseline (speedup 1.0000x reference)
import math

import jax
import jax.numpy as jnp
from jax import lax
from jax.experimental import pallas as pl
from jax.experimental.pallas import tpu as pltpu
from jax.experimental.pallas import tpu_sc as plsc

_EMB = 64
_B = 4096
_L = 200
_SCALE = math.sqrt(_EMB)

_NC = 2
_NS = 16
_NW = _NC * _NS

_ROWS_W = _B // _NW
_SPLIT = (96, 104)
_NCHUNK = _ROWS_W * 2
_NBUF = 4
_NOUTER = _NCHUNK // _NBUF


def _emb_body(tokens_hbm, table_hbm, out_hbm, idx_v, rows_in, rows_out, *sems):
    sem_g = sems[:_NBUF]
    sem_o = sems[_NBUF:]
    wid = lax.axis_index("s") * _NC + lax.axis_index("c")
    row0 = wid * _ROWS_W

    pltpu.sync_copy(tokens_hbm.at[pl.ds(row0, _ROWS_W)], idx_v)

    def chunk_geom(b):
        size = _SPLIT[b % 2]
        off = 0 if b % 2 == 0 else _SPLIT[0]
        return size, off

    def idx_chunk(c, b):
        size, off = chunk_geom(b)
        return idx_v.at[c // 2, pl.ds(off, size)]

    def out_chunk(c, b):
        size, off = chunk_geom(b)
        return out_hbm.at[row0 + c // 2, pl.ds(off, size)]

    def in_buf(b):
        return rows_in.at[b, pl.ds(0, chunk_geom(b)[0])]

    def out_buf(b):
        return rows_out.at[b, pl.ds(0, chunk_geom(b)[0])]

    for b in range(_NBUF):
        pltpu.make_async_copy(
            table_hbm.at[idx_chunk(b, b)], in_buf(b), sem_g[b]
        ).start()

    def round_body(g, carry):
        for b in range(_NBUF):
            c = g * _NBUF + b
            size, _ = chunk_geom(b)
            pltpu.make_async_copy(
                table_hbm.at[idx_chunk(c, b)], in_buf(b), sem_g[b]
            ).wait()

            @pl.when(g > 0)
            def _wait_out():
                pltpu.make_async_copy(out_buf(b), out_chunk(c, b), sem_o[b]).wait()

            def scale_row(r, acc):
                for j in range(4):
                    rows_out[b, r, pl.ds(j * 16, 16)] = (
                        rows_in[b, r, pl.ds(j * 16, 16)] * _SCALE
                    )
                return acc

            lax.fori_loop(0, size, scale_row, 0, unroll=8)

            pltpu.make_async_copy(out_buf(b), out_chunk(c, b), sem_o[b]).start()

            @pl.when(g < _NOUTER - 1)
            def _next_gather():
                pltpu.make_async_copy(
                    table_hbm.at[idx_chunk(c + _NBUF, b)], in_buf(b), sem_g[b]
                ).start()

        return carry

    lax.fori_loop(0, _NOUTER, round_body, 0)

    for b in range(_NBUF):
        pltpu.make_async_copy(
            out_buf(b), out_hbm.at[row0, pl.ds(chunk_geom(b)[1], chunk_geom(b)[0])], sem_o[b]
        ).wait()


@jax.jit
def _embed(tokens32, table):
    mesh = plsc.VectorSubcoreMesh(core_axis_name="c", subcore_axis_name="s")
    run = pl.kernel(
        _emb_body,
        out_type=jax.ShapeDtypeStruct((_B, _L, _EMB), jnp.float32),
        mesh=mesh,
        scratch_types=(
            [
                pltpu.VMEM((_ROWS_W, _L), jnp.int32),
                pltpu.VMEM((_NBUF, _SPLIT[1], _EMB), jnp.float32),
                pltpu.VMEM((_NBUF, _SPLIT[1], _EMB), jnp.float32),
            ]
            + [pltpu.SemaphoreType.DMA] * (2 * _NBUF)
        ),
        compiler_params=pltpu.CompilerParams(use_tc_tiling_on_sc=False),
    )
    return run(tokens32, table)


def kernel(tokens, table):
    return _embed(tokens.astype(jnp.int32), table)


# speedup vs baseline: 1.0706x; 1.0706x over previous
import math

import jax
import jax.numpy as jnp
from jax import lax
from jax.experimental import pallas as pl
from jax.experimental.pallas import tpu as pltpu
from jax.experimental.pallas import tpu_sc as plsc

_EMB = 64
_PAD = 128
_B = 4096
_L = 200
_SCALE = math.sqrt(_EMB)

_NC = 2
_NS = 16
_NW = _NC * _NS

_ROWS_W = _B // _NW
_TOK_W = _ROWS_W * _L
_SPLIT = (128, 72)
_NCHUNK = _ROWS_W * 2
_NBUF = 2
_NOUTER = _NCHUNK // _NBUF


def _emb_body(tokens_hbm, table_hbm, out_hbm, idx_v, rows, rows_out, *sems):
    sem_g = sems[:_NBUF]
    sem_o = sems[_NBUF:]
    wid = lax.axis_index("s") * _NC + lax.axis_index("c")
    row0 = wid * _ROWS_W

    pltpu.sync_copy(tokens_hbm.at[pl.ds(wid * _TOK_W, _TOK_W)], idx_v)

    def chunk_geom(b):
        size = _SPLIT[b % 2]
        off = 0 if b % 2 == 0 else _SPLIT[0]
        return size, off

    def idx_chunk(c, b):
        size, off = chunk_geom(b)
        return idx_v.at[pl.ds((c // 2) * _L + off, size)]

    def out_chunk(c, b):
        size, off = chunk_geom(b)
        return out_hbm.at[row0 + c // 2, pl.ds(off, size)]

    def in_buf(b):
        return rows.at[b, pl.ds(0, chunk_geom(b)[0])]

    def emb_buf(b):
        return rows_out.at[b, pl.ds(0, chunk_geom(b)[0])]

    for b in range(_NBUF):
        pltpu.make_async_copy(
            table_hbm.at[idx_chunk(b, b)], in_buf(b), sem_g[b]
        ).start()

    def round_body(g, carry):
        for b in range(_NBUF):
            c = g * _NBUF + b
            size, _ = chunk_geom(b)
            pltpu.make_async_copy(
                table_hbm.at[idx_chunk(c, b)], in_buf(b), sem_g[b]
            ).wait()

            @pl.when(g > 0)
            def _wait_out():
                pltpu.make_async_copy(emb_buf(b), out_chunk(c, b), sem_o[b]).wait()

            def scale_row(r, acc):
                for j in range(_EMB // 16):
                    rows_out[b, r, pl.ds(j * 16, 16)] = (
                        rows[b, r, pl.ds(j * 16, 16)] * _SCALE
                    )
                return acc

            lax.fori_loop(0, size, scale_row, 0, unroll=8)

            @pl.when(g < _NOUTER - 1)
            def _next_gather():
                pltpu.make_async_copy(
                    table_hbm.at[idx_chunk(c + _NBUF, b)], in_buf(b), sem_g[b]
                ).start()

            pltpu.make_async_copy(emb_buf(b), out_chunk(c, b), sem_o[b]).start()

        return carry

    lax.fori_loop(0, _NOUTER, round_body, 0)

    for b in range(_NBUF):
        size, off = chunk_geom(b)
        pltpu.make_async_copy(
            emb_buf(b), out_hbm.at[row0, pl.ds(off, size)], sem_o[b]
        ).wait()


@jax.jit
def _embed(tokens_flat, tablep):
    mesh = plsc.VectorSubcoreMesh(core_axis_name="c", subcore_axis_name="s")
    run = pl.kernel(
        _emb_body,
        out_type=jax.ShapeDtypeStruct((_B, _L, _EMB), jnp.float32),
        mesh=mesh,
        scratch_types=(
            [
                pltpu.VMEM((_TOK_W,), jnp.int32),
                pltpu.VMEM((_NBUF, _SPLIT[0], _PAD), jnp.float32),
                pltpu.VMEM((_NBUF, _SPLIT[0], _EMB), jnp.float32),
            ]
            + [pltpu.SemaphoreType.DMA] * (2 * _NBUF)
        ),
        compiler_params=pltpu.CompilerParams(use_tc_tiling_on_sc=True),
    )
    return run(tokens_flat, tablep)


def kernel(tokens, table):
    tablep = jnp.pad(table, ((0, 0), (0, _PAD - _EMB)))
    return _embed(tokens.astype(jnp.int32).reshape(-1), tablep)


# speedup vs baseline: 1.0797x; 1.0085x over previous
import math

import jax
import jax.numpy as jnp
from jax import lax
from jax.experimental import pallas as pl
from jax.experimental.pallas import tpu as pltpu
from jax.experimental.pallas import tpu_sc as plsc
from jax.experimental.layout import Format, Layout

_EMB = 64
_B = 4096
_L = 200
_SCALE = math.sqrt(_EMB)

_NC = 2
_NS = 16
_NW = _NC * _NS

_N = _B * _L
_PER_W = _N // _NW
_CHUNK = 128
_NCHUNK = _PER_W // _CHUNK
_NBUF = 4
_NOUTER = _NCHUNK // _NBUF


def _emb_body(tokens_hbm, table_hbm, out_hbm, idx_v, rows_in, rows_out, *sems):
    sem_g = sems[:_NBUF]
    sem_o = sems[_NBUF:]
    wid = lax.axis_index("s") * _NC + lax.axis_index("c")
    base = wid * _PER_W

    pltpu.sync_copy(tokens_hbm.at[wid], idx_v)

    for b in range(_NBUF):
        pltpu.make_async_copy(
            table_hbm.at[idx_v.at[b]], rows_in.at[b], sem_g[b]
        ).start()

    def round_body(g, carry):
        for b in range(_NBUF):
            c = g * _NBUF + b
            pltpu.make_async_copy(
                table_hbm.at[idx_v.at[c]], rows_in.at[b], sem_g[b]
            ).wait()

            @pl.when(g > 0)
            def _wait_out():
                pltpu.make_async_copy(
                    rows_out.at[b], out_hbm.at[pl.ds(base, _CHUNK)], sem_o[b]
                ).wait()

            def scale_row(r, acc):
                for j in range(4):
                    rows_out[b, r, pl.ds(j * 16, 16)] = (
                        rows_in[b, r, pl.ds(j * 16, 16)] * _SCALE
                    )
                return acc

            lax.fori_loop(0, _CHUNK, scale_row, 0, unroll=8)

            pltpu.make_async_copy(
                rows_out.at[b], out_hbm.at[pl.ds(base + c * _CHUNK, _CHUNK)], sem_o[b]
            ).start()

            @pl.when(g < _NOUTER - 1)
            def _next_gather():
                pltpu.make_async_copy(
                    table_hbm.at[idx_v.at[c + _NBUF]], rows_in.at[b], sem_g[b]
                ).start()

        return carry

    lax.fori_loop(0, _NOUTER, round_body, 0)

    for b in range(_NBUF):
        pltpu.make_async_copy(
            rows_out.at[b], out_hbm.at[pl.ds(base, _CHUNK)], sem_o[b]
        ).wait()


@jax.jit
def _embed(tokens32, table):
    mesh = plsc.VectorSubcoreMesh(core_axis_name="c", subcore_axis_name="s")
    run = pl.kernel(
        _emb_body,
        out_type=jax.ShapeDtypeStruct((_N, _EMB), jnp.float32),
        mesh=mesh,
        scratch_types=(
            [
                pltpu.VMEM((_NCHUNK, _CHUNK), jnp.int32),
                pltpu.VMEM((_NBUF, _CHUNK, _EMB), jnp.float32),
                pltpu.VMEM((_NBUF, _CHUNK, _EMB), jnp.float32),
            ]
            + [pltpu.SemaphoreType.DMA] * (2 * _NBUF)
        ),
        compiler_params=pltpu.CompilerParams(use_tc_tiling_on_sc=False),
    )
    return run(tokens32, table)


def kernel(tokens, table):
    sharding = jax.sharding.SingleDeviceSharding(jax.devices()[0])
    tablin = jax.device_put(
        table, Format(Layout(major_to_minor=(1, 0), tiling=((8,),)), sharding)
    )
    tok = tokens.astype(jnp.int32).reshape(_NW, _NCHUNK, _CHUNK)
    out = _embed(tok, tablin).reshape(_B, _L, _EMB)
    return jax.device_put(
        out, Format(Layout(major_to_minor=(0, 2, 1), tiling=((8, 128),)), sharding)
    )
